# baseline (speedup 1.0000x reference)
import functools

import jax
import jax.numpy as jnp
from jax import lax
from jax.experimental import pallas as pl
from jax.experimental.pallas import tpu as pltpu
from jax.experimental.pallas import tpu_sc as plsc

N = 10000
E = 320000
D = 128

NC = 2
NS = 16
NT = NC * NS
CHUNK = 40
EDGES_PER_TILE = E // NT
NCH = EDGES_PER_TILE // CHUNK
HALF = NCH // 2
NBUF = 4
ZR = CHUNK
NZCH = N // ZR


@functools.partial(
    pl.kernel,
    mesh=plsc.VectorSubcoreMesh(core_axis_name="c", subcore_axis_name="s"),
    out_type=jax.ShapeDtypeStruct((NC, N, D), jnp.float32),
    scratch_types=[
        pltpu.VMEM((HALF, CHUNK), jnp.int32),
        pltpu.VMEM((NBUF, CHUNK, D), jnp.float32),
        pltpu.VMEM_SHARED((N, D), jnp.float32),
        pltpu.SemaphoreType.DMA,
        pltpu.SemaphoreType.DMA,
        pltpu.SemaphoreType.DMA,
        pltpu.SemaphoreType.DMA,
        pltpu.SemaphoreType.DMA,
        pltpu.SemaphoreType.DMA,
        pltpu.SemaphoreType.DMA,
        pltpu.SemaphoreType.DMA,
        pltpu.SemaphoreType.DMA,
    ],
)
def _sc_agg(src_a, src_b, attr_hbm, out_hbm, idx_v, rows_v, agg_sh,
            l0, l1, l2, l3, s0, s1, s2, s3, isem):
    c = lax.axis_index("c")
    s = lax.axis_index("s")
    tile = c * NS + s
    lsem = (l0, l1, l2, l3)
    ssem = (s0, s1, s2, s3)
    base_edge = tile * EDGES_PER_TILE

    def load(p, j, b):
        pltpu.async_copy(
            attr_hbm.at[pl.ds(base_edge + p * HALF * CHUNK + j * CHUNK, CHUNK)],
            rows_v.at[b],
            lsem[b],
        )

    def wait_load(b):
        pltpu.make_async_copy(
            attr_hbm.at[pl.ds(base_edge, CHUNK)], rows_v.at[b], lsem[b]
        ).wait()

    def scatter(j, b):
        pltpu.async_copy(rows_v.at[b], agg_sh.at[idx_v.at[j]], ssem[b], add=True)

    def wait_scatter(j, b):
        pltpu.make_async_copy(rows_v.at[b], agg_sh.at[idx_v.at[j]], ssem[b]).wait()

    for b in range(2):
        load(0, b, b)
    pltpu.async_copy(src_a.at[tile], idx_v, isem)

    for r in range(ZR):
        for q in range(D // 16):
            rows_v[3, r, pl.ds(q * 16, 16)] = jnp.zeros((16,), jnp.float32)
    for t in range(NZCH // NS + 1):
        k = s + t * NS

        @pl.when(k < NZCH)
        def _():
            pltpu.sync_copy(rows_v.at[3], agg_sh.at[pl.ds(k * ZR, ZR)])

    pltpu.make_async_copy(src_a.at[tile], idx_v, isem).wait()
    plsc.subcore_barrier()

    for p in range(2):
        if p > 0:
            pltpu.sync_copy(src_b.at[tile], idx_v)
            for b in range(2):
                load(p, b, b)

        def _step(i, carry, p=p):
            for k in range(NBUF):
                j = i * NBUF + k
                b2 = (k + 2) % NBUF
                j2 = j + 2

                @pl.when((j >= 2) & (j2 < HALF))
                def _():
                    wait_scatter(j - 2, b2)

                @pl.when(j2 < HALF)
                def _():
                    load(p, j2, b2)

                @pl.when(j < HALF)
                def _():
                    wait_load(k)
                    scatter(j, k)

            return carry

        lax.fori_loop(0, (HALF + NBUF - 1) // NBUF, _step, 0)
        for jj in range(HALF - 4, HALF):
            wait_scatter(jj, jj % NBUF)

    plsc.subcore_barrier()

    for t in range(NZCH // NS + 1):
        k = s + t * NS

        @pl.when(k < NZCH)
        def _():
            pltpu.sync_copy(
                agg_sh.at[pl.ds(k * ZR, ZR)], out_hbm.at[c, pl.ds(k * ZR, ZR)]
            )


def _silu(h):
    return h * (1.0 / (1.0 + jnp.exp(-h)))


def _xa_body(x_ref, w_ref, b_ref, o_ref):
    o_ref[...] = (
        jnp.dot(x_ref[...], w_ref[...], precision=jax.lax.Precision.DEFAULT)
        + b_ref[...]
    )


_xa = pl.pallas_call(
    _xa_body,
    out_shape=jax.ShapeDtypeStruct((N, D), jnp.float32),
)


NB = 5
BLK = N // NB


def _mlp_body(xa_ref, p_ref, w1b_ref, g1_ref, bt1_ref,
              w2_ref, b2_ref, g2_ref, bt2_ref, w3_ref, b3_ref, o_ref,
              h_s, s_sum, s_sq, af_a, af_b):
    hp = jax.lax.Precision.DEFAULT
    g = pl.program_id(0)

    @pl.when(g < NB)
    def _():
        blk = g
        agg = p_ref[0] + p_ref[1]
        h = xa_ref[...] + jnp.dot(agg, w1b_ref[...], precision=hp)
        sil = _silu(h)
        h_s[pl.ds(blk * BLK, BLK), :] = sil

        @pl.when(g == 0)
        def _():
            s_sum[...] = jnp.zeros((1, D), jnp.float32)
            s_sq[...] = jnp.zeros((1, D), jnp.float32)

        s_sum[...] += jnp.sum(sil, axis=0, keepdims=True)
        s_sq[...] += jnp.sum(sil * sil, axis=0, keepdims=True)

    @pl.when((g >= NB) & (g < 2 * NB))
    def _():
        blk = g - NB

        @pl.when(g == NB)
        def _():
            mu = s_sum[...] * (1.0 / N)
            var = s_sq[...] * (1.0 / N) - mu * mu
            a = g1_ref[...] * jax.lax.rsqrt(var + 1e-5)
            af_a[...] = a
            af_b[...] = bt1_ref[...] - mu * a
            s_sum[...] = jnp.zeros((1, D), jnp.float32)
            s_sq[...] = jnp.zeros((1, D), jnp.float32)

        hn = h_s[pl.ds(blk * BLK, BLK), :] * af_a[...] + af_b[...]
        h2 = jnp.dot(hn, w2_ref[...], precision=hp) + b2_ref[...]
        sil = _silu(h2)
        h_s[pl.ds(blk * BLK, BLK), :] = sil
        s_sum[...] += jnp.sum(sil, axis=0, keepdims=True)
        s_sq[...] += jnp.sum(sil * sil, axis=0, keepdims=True)

    @pl.when(g >= 2 * NB)
    def _():
        blk = g - 2 * NB

        @pl.when(g == 2 * NB)
        def _():
            mu = s_sum[...] * (1.0 / N)
            var = s_sq[...] * (1.0 / N) - mu * mu
            a = g2_ref[...] * jax.lax.rsqrt(var + 1e-5)
            af_a[...] = a
            af_b[...] = bt2_ref[...] - mu * a

        hn = h_s[pl.ds(blk * BLK, BLK), :] * af_a[...] + af_b[...]
        o_ref[...] = jnp.dot(hn, w3_ref[...], precision=hp) + b3_ref[...]


def _row_blk(g):
    return (jnp.minimum(g, NB - 1), 0)


_mlp = pl.pallas_call(
    _mlp_body,
    grid=(3 * NB,),
    in_specs=[
        pl.BlockSpec((BLK, D), _row_blk),
        pl.BlockSpec((2, BLK, D), lambda g: (0, jnp.minimum(g, NB - 1), 0)),
        pl.BlockSpec((D, D), lambda g: (0, 0)),
        pl.BlockSpec((1, D), lambda g: (0, 0)),
        pl.BlockSpec((1, D), lambda g: (0, 0)),
        pl.BlockSpec((D, D), lambda g: (0, 0)),
        pl.BlockSpec((1, D), lambda g: (0, 0)),
        pl.BlockSpec((1, D), lambda g: (0, 0)),
        pl.BlockSpec((1, D), lambda g: (0, 0)),
        pl.BlockSpec((D, D), lambda g: (0, 0)),
        pl.BlockSpec((1, D), lambda g: (0, 0)),
    ],
    out_specs=pl.BlockSpec((BLK, D), lambda g: (jnp.maximum(g - 2 * NB, 0), 0)),
    out_shape=jax.ShapeDtypeStruct((N, D), jnp.float32),
    scratch_shapes=[
        pltpu.VMEM((N, D), jnp.float32),
        pltpu.VMEM((1, D), jnp.float32),
        pltpu.VMEM((1, D), jnp.float32),
        pltpu.VMEM((1, D), jnp.float32),
        pltpu.VMEM((1, D), jnp.float32),
    ],
)


def kernel(x, edge_index, edge_attr, W1, b1, g1, bt1, W2, b2, g2, bt2, W3, b3):
    src3d = edge_index[0].reshape(NT, NCH, CHUNK)
    xa = _xa(x, W1[:, :D].T, b1.reshape(1, D))
    parts = _sc_agg(src3d[:, :HALF], src3d[:, HALF:], edge_attr)
    return _mlp(
        xa, parts,
        W1[:, D:].T, g1.reshape(1, D), bt1.reshape(1, D),
        W2.T, b2.reshape(1, D), g2.reshape(1, D), bt2.reshape(1, D),
        W3.T, b3.reshape(1, D),
    )


# speedup vs baseline: 1.0439x; 1.0439x over previous
import functools

import jax
import jax.numpy as jnp
from jax import lax
from jax.experimental import pallas as pl
from jax.experimental.pallas import tpu as pltpu
from jax.experimental.pallas import tpu_sc as plsc

N = 10000
E = 320000
D = 128

NC = 2
NS = 16
NT = NC * NS
CHUNK = 40
EDGES_PER_TILE = E // NT
NCH = EDGES_PER_TILE // CHUNK
HALF = NCH // 2
NBUF = 4
ZR = CHUNK
NZCH = N // ZR


@functools.partial(
    pl.kernel,
    mesh=plsc.VectorSubcoreMesh(core_axis_name="c", subcore_axis_name="s"),
    out_type=jax.ShapeDtypeStruct((NC, N, D), jnp.float32),
    scratch_types=[
        pltpu.VMEM((HALF, CHUNK), jnp.int32),
        pltpu.VMEM((NBUF, CHUNK, D), jnp.float32),
        pltpu.VMEM_SHARED((N, D), jnp.float32),
        pltpu.SemaphoreType.DMA,
        pltpu.SemaphoreType.DMA,
        pltpu.SemaphoreType.DMA,
        pltpu.SemaphoreType.DMA,
        pltpu.SemaphoreType.DMA,
        pltpu.SemaphoreType.DMA,
        pltpu.SemaphoreType.DMA,
        pltpu.SemaphoreType.DMA,
        pltpu.SemaphoreType.DMA,
    ],
)
def _sc_agg(src_a, src_b, attr_hbm, out_hbm, idx_v, rows_v, agg_sh,
            l0, l1, l2, l3, s0, s1, s2, s3, isem):
    c = lax.axis_index("c")
    s = lax.axis_index("s")
    tile = c * NS + s
    lsem = (l0, l1, l2, l3)
    ssem = (s0, s1, s2, s3)
    base_edge = tile * EDGES_PER_TILE

    def load(p, j, b):
        pltpu.async_copy(
            attr_hbm.at[pl.ds(base_edge + p * HALF * CHUNK + j * CHUNK, CHUNK)],
            rows_v.at[b],
            lsem[b],
        )

    def wait_load(b):
        pltpu.make_async_copy(
            attr_hbm.at[pl.ds(base_edge, CHUNK)], rows_v.at[b], lsem[b]
        ).wait()

    def scatter(j, b):
        pltpu.async_copy(rows_v.at[b], agg_sh.at[idx_v.at[j]], ssem[b], add=True)

    def wait_scatter(j, b):
        pltpu.make_async_copy(rows_v.at[b], agg_sh.at[idx_v.at[j]], ssem[b]).wait()

    for b in range(2):
        load(0, b, b)
    pltpu.async_copy(src_a.at[tile], idx_v, isem)

    for r in range(ZR):
        for q in range(D // 16):
            rows_v[3, r, pl.ds(q * 16, 16)] = jnp.zeros((16,), jnp.float32)
    for t in range(NZCH // NS + 1):
        k = s + t * NS

        @pl.when(k < NZCH)
        def _():
            pltpu.async_copy(rows_v.at[3], agg_sh.at[pl.ds(k * ZR, ZR)], ssem[t % 4])

    for t in range(NZCH // NS + 1):
        k = s + t * NS

        @pl.when(k < NZCH)
        def _():
            pltpu.make_async_copy(
                rows_v.at[3], agg_sh.at[pl.ds(k * ZR, ZR)], ssem[t % 4]
            ).wait()

    pltpu.make_async_copy(src_a.at[tile], idx_v, isem).wait()
    plsc.subcore_barrier()

    for p in range(2):
        if p > 0:
            pltpu.sync_copy(src_b.at[tile], idx_v)
            for b in range(2):
                load(p, b, b)

        def _step(i, carry, p=p):
            for k in range(NBUF):
                j = i * NBUF + k
                b2 = (k + 2) % NBUF
                j2 = j + 2

                @pl.when((j >= 2) & (j2 < HALF))
                def _():
                    wait_scatter(j - 2, b2)

                @pl.when(j2 < HALF)
                def _():
                    load(p, j2, b2)

                @pl.when(j < HALF)
                def _():
                    wait_load(k)
                    scatter(j, k)

            return carry

        lax.fori_loop(0, (HALF + NBUF - 1) // NBUF, _step, 0)
        for jj in range(HALF - 4, HALF):
            wait_scatter(jj, jj % NBUF)

    plsc.subcore_barrier()

    for t in range(NZCH // NS + 1):
        k = s + t * NS

        @pl.when(k < NZCH)
        def _():
            pltpu.async_copy(
                agg_sh.at[pl.ds(k * ZR, ZR)], out_hbm.at[c, pl.ds(k * ZR, ZR)],
                lsem[t % 4],
            )

    for t in range(NZCH // NS + 1):
        k = s + t * NS

        @pl.when(k < NZCH)
        def _():
            pltpu.make_async_copy(
                agg_sh.at[pl.ds(k * ZR, ZR)], out_hbm.at[c, pl.ds(k * ZR, ZR)],
                lsem[t % 4],
            ).wait()


def _silu(h):
    return h * (1.0 / (1.0 + jnp.exp(-h)))


def _xa_body(x_ref, w_ref, b_ref, o_ref):
    o_ref[...] = (
        jnp.dot(x_ref[...], w_ref[...], precision=jax.lax.Precision.DEFAULT)
        + b_ref[...]
    )


_xa = pl.pallas_call(
    _xa_body,
    out_shape=jax.ShapeDtypeStruct((N, D), jnp.float32),
)


def _bn(h, g, bt):
    mu = jnp.mean(h, axis=0, keepdims=True)
    var = jnp.mean(h * h, axis=0, keepdims=True) - mu * mu
    a = g * jax.lax.rsqrt(var + 1e-5)
    return h * a + (bt - mu * a)


def _mlp_body(xa_ref, p_ref, w1b_ref, g1_ref, bt1_ref,
              w2_ref, b2_ref, g2_ref, bt2_ref, w3_ref, b3_ref, o_ref):
    hp = jax.lax.Precision.DEFAULT
    agg = p_ref[0] + p_ref[1]
    h = xa_ref[...] + jnp.dot(agg, w1b_ref[...], precision=hp)
    h = _bn(_silu(h), g1_ref[...], bt1_ref[...])
    h = jnp.dot(h, w2_ref[...], precision=hp) + b2_ref[...]
    h = _bn(_silu(h), g2_ref[...], bt2_ref[...])
    o_ref[...] = jnp.dot(h, w3_ref[...], precision=hp) + b3_ref[...]


_mlp = pl.pallas_call(
    _mlp_body,
    out_shape=jax.ShapeDtypeStruct((N, D), jnp.float32),
)


def kernel(x, edge_index, edge_attr, W1, b1, g1, bt1, W2, b2, g2, bt2, W3, b3):
    src3d = edge_index[0].reshape(NT, NCH, CHUNK)
    xa = _xa(x, W1[:, :D].T, b1.reshape(1, D))
    parts = _sc_agg(src3d[:, :HALF], src3d[:, HALF:], edge_attr)
    return _mlp(
        xa, parts,
        W1[:, D:].T, g1.reshape(1, D), bt1.reshape(1, D),
        W2.T, b2.reshape(1, D), g2.reshape(1, D), bt2.reshape(1, D),
        W3.T, b3.reshape(1, D),
    )
